# Initial kernel scaffold; baseline (speedup 1.0000x reference)
import jax
import jax.numpy as jnp
from jax.experimental import pallas as pl


def kernel(x, time, time_embedding, W, b):
    raise NotImplementedError("write your pallas kernel here")


# speedup vs baseline: 1.0418x; 1.0418x over previous
import functools

import jax
import jax.numpy as jnp
from jax import lax
from jax.experimental import pallas as pl
from jax.experimental.pallas import tpu as pltpu
from jax.experimental.pallas import tpu_sc as plsc

_IDX_CHUNK = 128


@functools.lru_cache(maxsize=None)
def _make_sc_gather(V, D, B):
    info = plsc.get_sparse_core_info()
    NW = info.num_cores * info.num_subcores
    NC = info.num_cores
    b_per_w = B // NW
    n_ch = b_per_w // _IDX_CHUNK
    mesh = plsc.VectorSubcoreMesh(core_axis_name="c", subcore_axis_name="s")

    @functools.partial(
        pl.kernel,
        out_type=jax.ShapeDtypeStruct((B, D), jnp.float32),
        mesh=mesh,
        scratch_types=[
            pltpu.VMEM((n_ch, _IDX_CHUNK), jnp.int32),
            pltpu.VMEM((b_per_w, D), jnp.float32),
            pltpu.SemaphoreType.DMA,
        ],
        compiler_params=pltpu.CompilerParams(use_tc_tiling_on_sc=False),
    )
    def gather_kernel(table_hbm, idx_hbm, out_hbm, idx_v, rows_v, sem):
        wid = lax.axis_index("s") * NC + lax.axis_index("c")
        base = wid * b_per_w
        pltpu.sync_copy(idx_hbm.at[pl.ds(wid * n_ch, n_ch)], idx_v)
        copies = [
            pltpu.async_copy(
                table_hbm.at[idx_v.at[j]],
                rows_v.at[pl.ds(j * _IDX_CHUNK, _IDX_CHUNK)],
                sem,
            )
            for j in range(n_ch)
        ]
        for c in copies:
            c.wait()
        pltpu.sync_copy(rows_v, out_hbm.at[pl.ds(base, b_per_w)])

    return gather_kernel


@functools.lru_cache(maxsize=None)
def _make_tc_matmul(B, DX, DT, DO, blk):
    def body(x_ref, t_ref, wx_ref, wt_ref, b_ref, o_ref):
        acc = jnp.dot(x_ref[...], wx_ref[...], preferred_element_type=jnp.float32)
        acc += jnp.dot(t_ref[...], wt_ref[...], preferred_element_type=jnp.float32)
        o_ref[...] = acc + b_ref[...]

    return pl.pallas_call(
        body,
        grid=(B // blk,),
        in_specs=[
            pl.BlockSpec((blk, DX), lambda i: (i, 0)),
            pl.BlockSpec((blk, DT), lambda i: (i, 0)),
            pl.BlockSpec((DX, DO), lambda i: (0, 0)),
            pl.BlockSpec((DT, DO), lambda i: (0, 0)),
            pl.BlockSpec((1, DO), lambda i: (0, 0)),
        ],
        out_specs=pl.BlockSpec((blk, DO), lambda i: (i, 0)),
        out_shape=jax.ShapeDtypeStruct((B, DO), jnp.float32),
    )


def kernel(x, time, time_embedding, W, b):
    B, DX = x.shape
    V, DT = time_embedding.shape
    DO = W.shape[1]
    idx = time.astype(jnp.int32).reshape(B // _IDX_CHUNK, _IDX_CHUNK)
    t = _make_sc_gather(V, DT, B)(time_embedding, idx)
    out = _make_tc_matmul(B, DX, DT, DO, 2048)(
        x, t, W[:DX], W[DX:], b.reshape(1, DO)
    )
    return out


# speedup vs baseline: 1.3502x; 1.2960x over previous
import functools

import jax
import jax.numpy as jnp
from jax import lax
from jax.experimental import pallas as pl
from jax.experimental.pallas import tpu as pltpu
from jax.experimental.pallas import tpu_sc as plsc

_IDX_CHUNK = 128


@functools.lru_cache(maxsize=None)
def _make_sc_gather(V, D, B):
    info = plsc.get_sparse_core_info()
    NW = info.num_cores * info.num_subcores
    NC = info.num_cores
    b_per_w = B // NW
    n_ch = b_per_w // _IDX_CHUNK
    mesh = plsc.VectorSubcoreMesh(core_axis_name="c", subcore_axis_name="s")

    @functools.partial(
        pl.kernel,
        out_type=jax.ShapeDtypeStruct((B, D), jnp.float32),
        mesh=mesh,
        scratch_types=[
            pltpu.VMEM((n_ch, _IDX_CHUNK), jnp.int32),
            pltpu.VMEM((b_per_w, D), jnp.float32),
            pltpu.VMEM_SHARED((V, D), jnp.float32),
            pltpu.SemaphoreType.DMA,
        ],
        compiler_params=pltpu.CompilerParams(use_tc_tiling_on_sc=False),
    )
    def gather_kernel(table_hbm, idx_hbm, out_hbm, idx_v, rows_v, tab_sp, sem):
        sid = lax.axis_index("s")
        wid = sid * NC + lax.axis_index("c")
        base = wid * b_per_w

        @pl.when(sid == 0)
        def _():
            pltpu.sync_copy(table_hbm, tab_sp)

        pltpu.sync_copy(idx_hbm.at[pl.ds(wid * n_ch, n_ch)], idx_v)
        plsc.subcore_barrier()
        copies = [
            pltpu.async_copy(
                tab_sp.at[idx_v.at[j]],
                rows_v.at[pl.ds(j * _IDX_CHUNK, _IDX_CHUNK)],
                sem,
            )
            for j in range(n_ch)
        ]
        for c in copies:
            c.wait()
        pltpu.sync_copy(rows_v, out_hbm.at[pl.ds(base, b_per_w)])

    return gather_kernel


@functools.lru_cache(maxsize=None)
def _make_tc_matmul(B, DX, DT, DO, blk):
    def body(x_ref, t_ref, wx_ref, wt_ref, b_ref, o_ref):
        acc = jnp.dot(x_ref[...], wx_ref[...], preferred_element_type=jnp.float32)
        acc += jnp.dot(t_ref[...], wt_ref[...], preferred_element_type=jnp.float32)
        o_ref[...] = acc + b_ref[...]

    return pl.pallas_call(
        body,
        grid=(B // blk,),
        in_specs=[
            pl.BlockSpec((blk, DX), lambda i: (i, 0)),
            pl.BlockSpec((blk, DT), lambda i: (i, 0)),
            pl.BlockSpec((DX, DO), lambda i: (0, 0)),
            pl.BlockSpec((DT, DO), lambda i: (0, 0)),
            pl.BlockSpec((1, DO), lambda i: (0, 0)),
        ],
        out_specs=pl.BlockSpec((blk, DO), lambda i: (i, 0)),
        out_shape=jax.ShapeDtypeStruct((B, DO), jnp.float32),
    )


def kernel(x, time, time_embedding, W, b):
    B, DX = x.shape
    V, DT = time_embedding.shape
    DO = W.shape[1]
    idx = time.astype(jnp.int32).reshape(B // _IDX_CHUNK, _IDX_CHUNK)
    t = _make_sc_gather(V, DT, B)(time_embedding, idx)
    out = _make_tc_matmul(B, DX, DT, DO, 2048)(
        x, t, W[:DX], W[DX:], b.reshape(1, DO)
    )
    return out


# speedup vs baseline: 1.4001x; 1.0369x over previous
import functools

import jax
import jax.numpy as jnp
from jax import lax
from jax.experimental import pallas as pl
from jax.experimental.pallas import tpu as pltpu
from jax.experimental.pallas import tpu_sc as plsc

_IDX_CHUNK = 128
_VPAD = 56


@functools.lru_cache(maxsize=None)
def _make_sc_gather_add(B, DO):
    info = plsc.get_sparse_core_info()
    NC = info.num_cores
    NW = NC * info.num_subcores
    b_per_w = B // NW
    n_ch = b_per_w // _IDX_CHUNK
    mesh = plsc.VectorSubcoreMesh(core_axis_name="c", subcore_axis_name="s")

    @functools.partial(
        pl.kernel,
        out_type=jax.ShapeDtypeStruct((B, DO), jnp.float32),
        mesh=mesh,
        scratch_types=[
            pltpu.VMEM((n_ch, _IDX_CHUNK), jnp.int32),
            pltpu.VMEM((b_per_w, DO), jnp.float32),
            pltpu.VMEM_SHARED((_VPAD, DO), jnp.float32),
            pltpu.SemaphoreType.DMA,
        ],
        compiler_params=pltpu.CompilerParams(use_tc_tiling_on_sc=False),
    )
    def gather_add_kernel(tabf_hbm, y_hbm, idx_hbm, out_hbm, idx_v, rows_v, tab_sp, sem):
        sid = lax.axis_index("s")
        wid = sid * NC + lax.axis_index("c")
        base = wid * b_per_w

        @pl.when(sid == 0)
        def _():
            pltpu.sync_copy(tabf_hbm, tab_sp)

        pltpu.sync_copy(idx_hbm.at[pl.ds(wid * n_ch, n_ch)], idx_v)
        pltpu.sync_copy(y_hbm.at[pl.ds(base, b_per_w)], rows_v)
        plsc.subcore_barrier()
        copies = [
            pltpu.async_copy(
                tab_sp.at[idx_v.at[j]],
                rows_v.at[pl.ds(j * _IDX_CHUNK, _IDX_CHUNK)],
                sem,
                add=True,
            )
            for j in range(n_ch)
        ]
        for c in copies:
            c.wait()
        pltpu.sync_copy(rows_v, out_hbm.at[pl.ds(base, b_per_w)])

    return gather_add_kernel


@functools.lru_cache(maxsize=None)
def _make_tc_dense(B, DX, DT, DO, blk):
    def body(x_ref, emb_ref, wx_ref, wt_ref, b_ref, y_ref, tabf_ref):
        @pl.when(pl.program_id(0) == 0)
        def _():
            tabf_ref[...] = jnp.dot(
                emb_ref[...], wt_ref[...], preferred_element_type=jnp.float32
            )

        y_ref[...] = (
            jnp.dot(x_ref[...], wx_ref[...], preferred_element_type=jnp.float32)
            + b_ref[...]
        )

    return pl.pallas_call(
        body,
        grid=(B // blk,),
        in_specs=[
            pl.BlockSpec((blk, DX), lambda i: (i, 0)),
            pl.BlockSpec((_VPAD, DT), lambda i: (0, 0)),
            pl.BlockSpec((DX, DO), lambda i: (0, 0)),
            pl.BlockSpec((DT, DO), lambda i: (0, 0)),
            pl.BlockSpec((1, DO), lambda i: (0, 0)),
        ],
        out_specs=[
            pl.BlockSpec((blk, DO), lambda i: (i, 0)),
            pl.BlockSpec((_VPAD, DO), lambda i: (0, 0)),
        ],
        out_shape=[
            jax.ShapeDtypeStruct((B, DO), jnp.float32),
            jax.ShapeDtypeStruct((_VPAD, DO), jnp.float32),
        ],
    )


def kernel(x, time, time_embedding, W, b):
    B, DX = x.shape
    V, DT = time_embedding.shape
    DO = W.shape[1]
    emb_pad = jnp.pad(time_embedding, ((0, _VPAD - V), (0, 0)))
    idx = time.astype(jnp.int32).reshape(B // _IDX_CHUNK, _IDX_CHUNK)
    y, tabf = _make_tc_dense(B, DX, DT, DO, 2048)(
        x, emb_pad, W[:DX], W[DX:], b.reshape(1, DO)
    )
    return _make_sc_gather_add(B, DO)(tabf, y, idx)


# speedup vs baseline: 1.4547x; 1.0390x over previous
import functools

import jax
import jax.numpy as jnp
from jax import lax
from jax.experimental import pallas as pl
from jax.experimental.pallas import tpu as pltpu
from jax.experimental.pallas import tpu_sc as plsc

_IDX_CHUNK = 128


@functools.lru_cache(maxsize=None)
def _make_sc_gather(V, D, B):
    info = plsc.get_sparse_core_info()
    NC = info.num_cores
    NW = NC * info.num_subcores
    b_per_w = B // NW
    n_ch = b_per_w // _IDX_CHUNK
    mesh = plsc.VectorSubcoreMesh(core_axis_name="c", subcore_axis_name="s")

    @functools.partial(
        pl.kernel,
        out_type=jax.ShapeDtypeStruct((B, D), jnp.float32),
        mesh=mesh,
        scratch_types=[
            pltpu.VMEM((n_ch, _IDX_CHUNK), jnp.int32),
            pltpu.VMEM((b_per_w, D), jnp.float32),
            pltpu.VMEM_SHARED((V, D), jnp.float32),
            pltpu.SemaphoreType.DMA,
        ],
        compiler_params=pltpu.CompilerParams(use_tc_tiling_on_sc=False),
    )
    def gather_kernel(table_hbm, idx_hbm, out_hbm, idx_v, rows_v, tab_sp, sem):
        sid = lax.axis_index("s")
        wid = sid * NC + lax.axis_index("c")
        base = wid * b_per_w

        @pl.when(sid == 0)
        def _():
            pltpu.sync_copy(table_hbm, tab_sp)

        pltpu.sync_copy(idx_hbm.at[pl.ds(wid * n_ch, n_ch)], idx_v)
        plsc.subcore_barrier()
        copies = [
            pltpu.async_copy(
                tab_sp.at[idx_v.at[j]],
                rows_v.at[pl.ds(j * _IDX_CHUNK, _IDX_CHUNK)],
                sem,
            )
            for j in range(n_ch)
        ]
        for c in copies:
            c.wait()
        pltpu.sync_copy(rows_v, out_hbm.at[pl.ds(base, b_per_w)])

    return gather_kernel


@functools.lru_cache(maxsize=None)
def _make_tc_matmul(B, DX, DT, DO, blk):
    def body(x_ref, t_ref, wx_ref, wt_ref, b_ref, o_ref):
        acc = jnp.dot(x_ref[...], wx_ref[...], preferred_element_type=jnp.float32)
        acc += jnp.dot(t_ref[...], wt_ref[...], preferred_element_type=jnp.float32)
        o_ref[...] = acc + b_ref[...]

    return pl.pallas_call(
        body,
        grid=(B // blk,),
        in_specs=[
            pl.BlockSpec((blk, DX), lambda i: (i, 0)),
            pl.BlockSpec((blk, DT), lambda i: (i, 0)),
            pl.BlockSpec((DX, DO), lambda i: (0, 0)),
            pl.BlockSpec((DT, DO), lambda i: (0, 0)),
            pl.BlockSpec((1, DO), lambda i: (0, 0)),
        ],
        out_specs=pl.BlockSpec((blk, DO), lambda i: (i, 0)),
        out_shape=jax.ShapeDtypeStruct((B, DO), jnp.float32),
    )


def kernel(x, time, time_embedding, W, b):
    B, DX = x.shape
    V, DT = time_embedding.shape
    DO = W.shape[1]
    idx = time.astype(jnp.int32).reshape(B // _IDX_CHUNK, _IDX_CHUNK)
    t = _make_sc_gather(V, DT, B)(time_embedding, idx)
    return _make_tc_matmul(B, DX, DT, DO, 8192)(
        x, t, W[:DX], W[DX:], b.reshape(1, DO)
    )


# speedup vs baseline: 1.4578x; 1.0021x over previous
import functools

import jax
import jax.numpy as jnp
from jax import lax
from jax.experimental import pallas as pl
from jax.experimental.pallas import tpu as pltpu
from jax.experimental.pallas import tpu_sc as plsc

_IDX_CHUNK = 128


@functools.lru_cache(maxsize=None)
def _make_sc_gather(V, D, B):
    info = plsc.get_sparse_core_info()
    NC = info.num_cores
    NW = NC * info.num_subcores
    b_per_w = B // NW
    n_ch = b_per_w // _IDX_CHUNK
    mesh = plsc.VectorSubcoreMesh(core_axis_name="c", subcore_axis_name="s")

    @functools.partial(
        pl.kernel,
        out_type=jax.ShapeDtypeStruct((B, D), jnp.float32),
        mesh=mesh,
        scratch_types=[
            pltpu.VMEM((b_per_w,), jnp.int32),
            pltpu.VMEM((b_per_w, D), jnp.float32),
            pltpu.VMEM_SHARED((V, D), jnp.float32),
            pltpu.SemaphoreType.DMA,
        ],
        compiler_params=pltpu.CompilerParams(use_tc_tiling_on_sc=False),
    )
    def gather_kernel(table_hbm, idx_hbm, out_hbm, idx_v, rows_v, tab_sp, sem):
        sid = lax.axis_index("s")
        wid = sid * NC + lax.axis_index("c")
        base = wid * b_per_w

        @pl.when(sid == 0)
        def _():
            pltpu.sync_copy(table_hbm, tab_sp)

        pltpu.sync_copy(idx_hbm.at[pl.ds(base, b_per_w)], idx_v)
        plsc.subcore_barrier()
        copies = [
            pltpu.async_copy(
                tab_sp.at[idx_v.at[pl.ds(j * _IDX_CHUNK, _IDX_CHUNK)]],
                rows_v.at[pl.ds(j * _IDX_CHUNK, _IDX_CHUNK)],
                sem,
            )
            for j in range(n_ch)
        ]
        for c in copies:
            c.wait()
        pltpu.sync_copy(rows_v, out_hbm.at[pl.ds(base, b_per_w)])

    return gather_kernel


@functools.lru_cache(maxsize=None)
def _make_tc_matmul(B, DX, DT, DO, blk):
    def body(x_ref, t_ref, w_ref, b_ref, o_ref):
        acc = jnp.dot(
            x_ref[...], w_ref[:DX, :], preferred_element_type=jnp.float32
        )
        acc += jnp.dot(
            t_ref[...], w_ref[DX:, :], preferred_element_type=jnp.float32
        )
        o_ref[...] = acc + b_ref[...]

    return pl.pallas_call(
        body,
        grid=(B // blk,),
        in_specs=[
            pl.BlockSpec((blk, DX), lambda i: (i, 0)),
            pl.BlockSpec((blk, DT), lambda i: (i, 0)),
            pl.BlockSpec((DX + DT, DO), lambda i: (0, 0)),
            pl.BlockSpec((1, DO), lambda i: (0, 0)),
        ],
        out_specs=pl.BlockSpec((blk, DO), lambda i: (i, 0)),
        out_shape=jax.ShapeDtypeStruct((B, DO), jnp.float32),
    )


def kernel(x, time, time_embedding, W, b):
    B, DX = x.shape
    V, DT = time_embedding.shape
    DO = W.shape[1]
    t = _make_sc_gather(V, DT, B)(time_embedding, time.astype(jnp.int32))
    return _make_tc_matmul(B, DX, DT, DO, 8192)(x, t, W, b.reshape(1, DO))
